# baseline (speedup 1.0000x reference)
import functools

import jax
import jax.numpy as jnp
from jax import lax
from jax.experimental import pallas as pl
from jax.experimental.pallas import tpu as pltpu
from jax.experimental.pallas import tpu_sc as plsc

N = 10000
HD = 128
CLS = 40
E = 320000

NC = 2
NS = 16
NTILE = NC * NS
CH = 128
NJ = 80
EPT = NJ * CH
EP = NTILE * EPT
GR = N
NACC = 10240
SLAB = NACC // NS
TSL = 624
DW = 16
NBUF = 2

BR = 2000
_mesh = plsc.VectorSubcoreMesh(core_axis_name="core", subcore_axis_name="subcore")


@functools.partial(
    pl.kernel,
    out_type=jax.ShapeDtypeStruct((NC, NACC, DW), jnp.float32),
    mesh=_mesh,
    scratch_types=[
        pltpu.VMEM((NJ, CH), jnp.int32),
        pltpu.VMEM((CH, DW), jnp.float32),
        pltpu.VMEM((CH, DW), jnp.float32),
        pltpu.VMEM_SHARED((NACC, DW), jnp.float32),
        pltpu.SemaphoreType.DMA,
    ],
)
def _deg_sc(didx_hbm, out_hbm, didx_v, ones_v, zero_v, acc_sh, isem):
    c = lax.axis_index("core")
    s = lax.axis_index("subcore")
    wid = c * NS + s

    pltpu.async_copy(didx_hbm.at[wid], didx_v, isem)

    @pl.loop(0, CH)
    def _(r):
        ones_v[r, :] = jnp.ones((DW,), jnp.float32)
        zero_v[r, :] = jnp.zeros((DW,), jnp.float32)

    base = s * SLAB

    @pl.loop(0, SLAB // CH)
    def _(k):
        pltpu.sync_copy(zero_v, acc_sh.at[pl.ds(base + k * CH, CH)])

    pltpu.make_async_copy(didx_hbm.at[wid], didx_v, isem).wait()
    plsc.subcore_barrier()

    @pl.loop(0, NJ)
    def _(j):
        pltpu.sync_copy(ones_v, acc_sh.at[didx_v.at[j]], add=True)

    plsc.subcore_barrier()
    pltpu.sync_copy(acc_sh.at[pl.ds(base, SLAB)], out_hbm.at[c, pl.ds(base, SLAB)])


@functools.partial(
    pl.kernel,
    out_type=jax.ShapeDtypeStruct((EP, HD), jnp.float32),
    mesh=_mesh,
    scratch_types=[
        pltpu.VMEM((NJ, CH), jnp.int32),
        [pltpu.VMEM((CH, HD), jnp.float32)] * NBUF,
        pltpu.VMEM_SHARED((N, HD), jnp.float32),
        [pltpu.SemaphoreType.DMA] * NBUF,
        [pltpu.SemaphoreType.DMA] * NBUF,
        pltpu.SemaphoreType.DMA,
    ],
)
def _gcn_expand_sc(table_hbm, sidx_hbm, msg_hbm,
                   sidx_v, bufs, tab_sh, gsems, wsems, isem):
    c = lax.axis_index("core")
    s = lax.axis_index("subcore")
    wid = c * NS + s
    ebase = wid * EPT

    pltpu.async_copy(sidx_hbm.at[wid], sidx_v, isem)

    tb = s * TSL
    pltpu.sync_copy(table_hbm.at[pl.ds(tb, TSL)], tab_sh.at[pl.ds(tb, TSL)])

    @pl.when(s == 0)
    def _():
        pltpu.sync_copy(table_hbm.at[pl.ds(NS * TSL, N - NS * TSL)],
                        tab_sh.at[pl.ds(NS * TSL, N - NS * TSL)])

    pltpu.make_async_copy(sidx_hbm.at[wid], sidx_v, isem).wait()
    plsc.subcore_barrier()

    def _gather(j, b):
        pltpu.async_copy(tab_sh.at[sidx_v.at[j]], bufs[b], gsems[b])

    def _wait_gather(j, b):
        pltpu.make_async_copy(tab_sh.at[sidx_v.at[j]], bufs[b], gsems[b]).wait()

    def _write(j, b):
        pltpu.async_copy(bufs[b], msg_hbm.at[pl.ds(ebase + j * CH, CH)], wsems[b])

    def _wait_write(j, b):
        pltpu.make_async_copy(bufs[b], msg_hbm.at[pl.ds(ebase + j * CH, CH)],
                              wsems[b]).wait()

    _gather(0, 0)

    @pl.loop(0, NJ, step=NBUF)
    def _(j):
        for b in range(NBUF):
            jj = j + b
            nb = (b + 1) % NBUF

            @pl.when(jj >= 1)
            def _():
                _wait_write(jj - 1, nb)

            @pl.when(jj + 1 < NJ)
            def _():
                _gather(jj + 1, nb)

            _wait_gather(jj, b)
            _write(jj, b)

    _wait_write(NJ - 1, (NJ - 1) % NBUF)


@functools.partial(
    pl.kernel,
    out_type=jax.ShapeDtypeStruct((NC, NACC, HD), jnp.float32),
    mesh=_mesh,
    scratch_types=[
        pltpu.VMEM((NJ, CH), jnp.int32),
        [pltpu.VMEM((CH, HD), jnp.float32)] * NBUF,
        pltpu.VMEM_SHARED((NACC, HD), jnp.float32),
        [pltpu.SemaphoreType.DMA] * NBUF,
        pltpu.SemaphoreType.DMA,
    ],
)
def _gcn_reduce_sc(msg_hbm, didx_hbm, out_hbm,
                   didx_v, bufs, acc_sh, rsems, isem):
    c = lax.axis_index("core")
    s = lax.axis_index("subcore")
    wid = c * NS + s
    ebase = wid * EPT

    pltpu.async_copy(didx_hbm.at[wid], didx_v, isem)

    @pl.loop(0, CH)
    def _(r):
        @pl.loop(0, HD, step=16)
        def _(col):
            bufs[0][r, pl.ds(col, 16)] = jnp.zeros((16,), jnp.float32)

    base = s * SLAB

    @pl.loop(0, SLAB // CH)
    def _(k):
        pltpu.sync_copy(bufs[0], acc_sh.at[pl.ds(base + k * CH, CH)])

    pltpu.make_async_copy(didx_hbm.at[wid], didx_v, isem).wait()
    plsc.subcore_barrier()

    def _read(j, b):
        pltpu.async_copy(msg_hbm.at[pl.ds(ebase + j * CH, CH)], bufs[b], rsems[b])

    def _wait_read(j, b):
        pltpu.make_async_copy(msg_hbm.at[pl.ds(ebase + j * CH, CH)],
                              bufs[b], rsems[b]).wait()

    def _scatter(j, b):
        pltpu.sync_copy(bufs[b], acc_sh.at[didx_v.at[j]], add=True)

    _read(0, 0)

    @pl.loop(0, NJ, step=NBUF)
    def _(j):
        for b in range(NBUF):
            jj = j + b
            nb = (b + 1) % NBUF

            @pl.when(jj + 1 < NJ)
            def _():
                _read(jj + 1, nb)

            _wait_read(jj, b)
            _scatter(jj, b)

    plsc.subcore_barrier()
    pltpu.sync_copy(acc_sh.at[pl.ds(base, SLAB)], out_hbm.at[c, pl.ds(base, SLAB)])


def _gcn_agg(hw, sidx, didx):
    msg = _gcn_expand_sc(hw, sidx)
    return _gcn_reduce_sc(msg, didx)


def _gelu(x):
    return 0.5 * x * (1.0 + lax.erf(x * 0.7071067811865476))


def _dinv_of(degp_ref):
    deg = degp_ref[0] + degp_ref[1] + 1.0
    return lax.rsqrt(deg[:, 0:1])


def _tc1_body(x_ref, wred_ref, bred_ref, w1_ref, degp_ref, h0_ref, hw1_ref):
    h0 = _gelu(jnp.dot(x_ref[...], wred_ref[...],
                       preferred_element_type=jnp.float32) + bred_ref[...])
    dinv = _dinv_of(degp_ref)
    h0_ref[...] = h0
    hw1_ref[...] = jnp.dot(h0, w1_ref[...],
                           preferred_element_type=jnp.float32) * dinv


def _post_conv(acc_ref, hw_ref, res_ref, dinv, b_ref, g_ref, be_ref, m_ref, v_ref):
    agg = acc_ref[0] + acc_ref[1] + hw_ref[...]
    conv = agg * dinv + b_ref[...]
    bn = (conv - m_ref[...]) * lax.rsqrt(v_ref[...] + 1e-5) * g_ref[...] + be_ref[...]
    return _gelu(bn) + res_ref[...]


def _tc2_body(acc_ref, hw_ref, res_ref, degp_ref, b_ref, g_ref, be_ref,
              m_ref, v_ref, w_ref, h_ref, hwn_ref):
    dinv = _dinv_of(degp_ref)
    h = _post_conv(acc_ref, hw_ref, res_ref, dinv, b_ref, g_ref, be_ref, m_ref, v_ref)
    h_ref[...] = h
    hwn_ref[...] = jnp.dot(h, w_ref[...], preferred_element_type=jnp.float32) * dinv


def _tc3_body(acc_ref, hw_ref, res_ref, degp_ref, b_ref, g_ref, be_ref,
              m_ref, v_ref, wlin_ref, blin_ref, out_ref):
    dinv = _dinv_of(degp_ref)
    h = _post_conv(acc_ref, hw_ref, res_ref, dinv, b_ref, g_ref, be_ref, m_ref, v_ref)
    out_ref[...] = jnp.dot(h, wlin_ref[...],
                           preferred_element_type=jnp.float32) + blin_ref[...]


_row_spec = pl.BlockSpec((BR, HD), lambda i: (i, 0))
_w_spec = pl.BlockSpec((HD, HD), lambda i: (0, 0))
_vec_spec = pl.BlockSpec((1, HD), lambda i: (0, 0))
_deg_spec = pl.BlockSpec((2, BR, DW), lambda i: (0, i, 0))
_acc_spec = pl.BlockSpec((2, BR, HD), lambda i: (0, i, 0))
_G = N // BR


def _tc1(x, wred, bred, w1, degp):
    return pl.pallas_call(
        _tc1_body,
        grid=(_G,),
        in_specs=[_row_spec, _w_spec, _vec_spec, _w_spec, _deg_spec],
        out_specs=[_row_spec, _row_spec],
        out_shape=[jax.ShapeDtypeStruct((N, HD), jnp.float32)] * 2,
    )(x, wred, bred, w1, degp)


def _tc2(acc, hw, res, degp, b, g, be, m, v, w):
    return pl.pallas_call(
        _tc2_body,
        grid=(_G,),
        in_specs=[_acc_spec, _row_spec, _row_spec, _deg_spec,
                  _vec_spec, _vec_spec, _vec_spec, _vec_spec, _vec_spec, _w_spec],
        out_specs=[_row_spec, _row_spec],
        out_shape=[jax.ShapeDtypeStruct((N, HD), jnp.float32)] * 2,
    )(acc, hw, res, degp, b, g, be, m, v, w)


def _tc3(acc, hw, res, degp, b, g, be, m, v, wlin, blin):
    return pl.pallas_call(
        _tc3_body,
        grid=(_G,),
        in_specs=[_acc_spec, _row_spec, _row_spec, _deg_spec,
                  _vec_spec, _vec_spec, _vec_spec, _vec_spec, _vec_spec,
                  pl.BlockSpec((HD, CLS), lambda i: (0, 0)),
                  pl.BlockSpec((1, CLS), lambda i: (0, 0))],
        out_specs=[pl.BlockSpec((BR, CLS), lambda i: (i, 0))],
        out_shape=[jax.ShapeDtypeStruct((N, CLS), jnp.float32)],
    )(acc, hw, res, degp, b, g, be, m, v, wlin, blin)[0]


def kernel(x, edge_index, W_red, b_red, W1, b1, g1, beta1, m1, v1,
           W2, b2, g2, beta2, m2, v2, W_lin, b_lin):
    src = edge_index[0]
    dst = edge_index[1]
    sidx = jnp.concatenate([src, jnp.zeros((EP - E,), jnp.int32)])
    didx = jnp.concatenate([dst, jnp.full((EP - E,), GR, jnp.int32)])
    sidx = sidx.reshape(NTILE, NJ, CH)
    didx = didx.reshape(NTILE, NJ, CH)

    degp = _deg_sc(didx)
    h0, hw1 = _tc1(x, W_red, b_red.reshape(1, HD), W1, degp)
    acc1 = _gcn_agg(hw1, sidx, didx)
    h1, hw2 = _tc2(acc1, hw1, h0, degp, b1.reshape(1, HD), g1.reshape(1, HD),
                   beta1.reshape(1, HD), m1.reshape(1, HD), v1.reshape(1, HD), W2)
    acc2 = _gcn_agg(hw2, sidx, didx)
    return _tc3(acc2, hw2, h1, degp, b2.reshape(1, HD), g2.reshape(1, HD),
                beta2.reshape(1, HD), m2.reshape(1, HD), v2.reshape(1, HD),
                W_lin, b_lin.reshape(1, CLS))


# speedup vs baseline: 1.0079x; 1.0079x over previous
import functools

import jax
import jax.numpy as jnp
from jax import lax
from jax.experimental import pallas as pl
from jax.experimental.pallas import tpu as pltpu
from jax.experimental.pallas import tpu_sc as plsc

N = 10000
HD = 128
CLS = 40
E = 320000

NC = 2
NS = 16
NTILE = NC * NS
CH = 128
NJ = 80
EPT = NJ * CH
EP = NTILE * EPT
GR = N
NACC = 10240
SLAB = NACC // NS
TSL = 624
DW = 16
NBUF = 2

BR = 2000
_mesh = plsc.VectorSubcoreMesh(core_axis_name="core", subcore_axis_name="subcore")


@functools.partial(
    pl.kernel,
    out_type=jax.ShapeDtypeStruct((NC, NACC, DW), jnp.float32),
    mesh=_mesh,
    scratch_types=[
        pltpu.VMEM((NJ, CH), jnp.int32),
        pltpu.VMEM((CH, DW), jnp.float32),
        pltpu.VMEM((CH, DW), jnp.float32),
        pltpu.VMEM_SHARED((NACC, DW), jnp.float32),
        pltpu.SemaphoreType.DMA,
    ],
)
def _deg_sc(didx_hbm, out_hbm, didx_v, ones_v, zero_v, acc_sh, isem):
    c = lax.axis_index("core")
    s = lax.axis_index("subcore")
    wid = c * NS + s

    pltpu.async_copy(didx_hbm.at[wid], didx_v, isem)

    @pl.loop(0, CH)
    def _(r):
        ones_v[r, :] = jnp.ones((DW,), jnp.float32)
        zero_v[r, :] = jnp.zeros((DW,), jnp.float32)

    base = s * SLAB

    @pl.loop(0, SLAB // CH)
    def _(k):
        pltpu.sync_copy(zero_v, acc_sh.at[pl.ds(base + k * CH, CH)])

    pltpu.make_async_copy(didx_hbm.at[wid], didx_v, isem).wait()
    plsc.subcore_barrier()

    @pl.loop(0, NJ)
    def _(j):
        pltpu.sync_copy(ones_v, acc_sh.at[didx_v.at[j]], add=True)

    plsc.subcore_barrier()
    pltpu.sync_copy(acc_sh.at[pl.ds(base, SLAB)], out_hbm.at[c, pl.ds(base, SLAB)])


@functools.partial(
    pl.kernel,
    out_type=jax.ShapeDtypeStruct((EP, HD), jnp.float32),
    mesh=_mesh,
    scratch_types=[
        pltpu.VMEM((NJ, CH), jnp.int32),
        [pltpu.VMEM((CH, HD), jnp.float32)] * NBUF,
        pltpu.VMEM_SHARED((N, HD), jnp.float32),
        [pltpu.SemaphoreType.DMA] * NBUF,
        [pltpu.SemaphoreType.DMA] * NBUF,
        pltpu.SemaphoreType.DMA,
    ],
)
def _gcn_expand_sc(table_hbm, sidx_hbm, msg_hbm,
                   sidx_v, bufs, tab_sh, gsems, wsems, isem):
    c = lax.axis_index("core")
    s = lax.axis_index("subcore")
    wid = c * NS + s
    ebase = wid * EPT

    pltpu.async_copy(sidx_hbm.at[wid], sidx_v, isem)

    tb = s * TSL
    pltpu.sync_copy(table_hbm.at[pl.ds(tb, TSL)], tab_sh.at[pl.ds(tb, TSL)])

    @pl.when(s == 0)
    def _():
        pltpu.sync_copy(table_hbm.at[pl.ds(NS * TSL, N - NS * TSL)],
                        tab_sh.at[pl.ds(NS * TSL, N - NS * TSL)])

    pltpu.make_async_copy(sidx_hbm.at[wid], sidx_v, isem).wait()
    plsc.subcore_barrier()

    def _gather(j, b):
        pltpu.async_copy(tab_sh.at[sidx_v.at[j]], bufs[b], gsems[b])

    def _wait_gather(j, b):
        pltpu.make_async_copy(tab_sh.at[sidx_v.at[j]], bufs[b], gsems[b]).wait()

    def _write(j, b):
        pltpu.async_copy(bufs[b], msg_hbm.at[pl.ds(ebase + j * CH, CH)], wsems[b])

    def _wait_write(j, b):
        pltpu.make_async_copy(bufs[b], msg_hbm.at[pl.ds(ebase + j * CH, CH)],
                              wsems[b]).wait()

    _gather(0, 0)

    @pl.loop(0, NJ, step=NBUF)
    def _(j):
        for b in range(NBUF):
            jj = j + b
            nb = (b + 1) % NBUF

            @pl.when(jj >= 1)
            def _():
                _wait_write(jj - 1, nb)

            @pl.when(jj + 1 < NJ)
            def _():
                _gather(jj + 1, nb)

            _wait_gather(jj, b)
            _write(jj, b)

    _wait_write(NJ - 1, (NJ - 1) % NBUF)


@functools.partial(
    pl.kernel,
    out_type=jax.ShapeDtypeStruct((NC, NACC, HD), jnp.float32),
    mesh=_mesh,
    scratch_types=[
        pltpu.VMEM((NJ, CH), jnp.int32),
        [pltpu.VMEM((CH, HD), jnp.float32)] * NBUF,
        pltpu.VMEM_SHARED((NACC, HD), jnp.float32),
        [pltpu.SemaphoreType.DMA] * NBUF,
        pltpu.SemaphoreType.DMA,
    ],
)
def _gcn_reduce_sc(msg_hbm, didx_hbm, out_hbm,
                   didx_v, bufs, acc_sh, rsems, isem):
    c = lax.axis_index("core")
    s = lax.axis_index("subcore")
    wid = c * NS + s
    ebase = wid * EPT

    pltpu.async_copy(didx_hbm.at[wid], didx_v, isem)

    @pl.loop(0, CH)
    def _(r):
        @pl.loop(0, HD, step=16)
        def _(col):
            bufs[0][r, pl.ds(col, 16)] = jnp.zeros((16,), jnp.float32)

    base = s * SLAB

    @pl.loop(0, SLAB // CH)
    def _(k):
        pltpu.sync_copy(bufs[0], acc_sh.at[pl.ds(base + k * CH, CH)])

    pltpu.make_async_copy(didx_hbm.at[wid], didx_v, isem).wait()
    plsc.subcore_barrier()

    def _read(j, b):
        pltpu.async_copy(msg_hbm.at[pl.ds(ebase + j * CH, CH)], bufs[b], rsems[b])

    def _wait_read(j, b):
        pltpu.make_async_copy(msg_hbm.at[pl.ds(ebase + j * CH, CH)],
                              bufs[b], rsems[b]).wait()

    def _scatter(j, b):
        pltpu.sync_copy(bufs[b], acc_sh.at[didx_v.at[j]], add=True)

    _read(0, 0)

    @pl.loop(0, NJ, step=NBUF)
    def _(j):
        for b in range(NBUF):
            jj = j + b
            nb = (b + 1) % NBUF

            @pl.when(jj + 1 < NJ)
            def _():
                _read(jj + 1, nb)

            _wait_read(jj, b)
            _scatter(jj, b)

    plsc.subcore_barrier()
    pltpu.sync_copy(acc_sh.at[pl.ds(base, SLAB)], out_hbm.at[c, pl.ds(base, SLAB)])


def _gcn_agg(hw, sidx, didx):
    msg = _gcn_expand_sc(hw, sidx)
    return _gcn_reduce_sc(msg, didx)


def _gelu(x):
    return 0.5 * x * (1.0 + lax.erf(x * 0.7071067811865476))


def _dinv_of(degp_ref):
    deg = degp_ref[0] + degp_ref[1] + 1.0
    return lax.rsqrt(deg[:, 0:1])


def _tc0_body(x_ref, wred_ref, bred_ref, h0_ref):
    h0_ref[...] = _gelu(jnp.dot(x_ref[...], wred_ref[...],
                                preferred_element_type=jnp.float32) + bred_ref[...])


def _tc1_body(h0_ref, w1_ref, degp_ref, hw1_ref):
    dinv = _dinv_of(degp_ref)
    hw1_ref[...] = jnp.dot(h0_ref[...], w1_ref[...],
                           preferred_element_type=jnp.float32) * dinv


def _post_conv(acc_ref, hw_ref, res_ref, dinv, b_ref, g_ref, be_ref, m_ref, v_ref):
    agg = acc_ref[0] + acc_ref[1] + hw_ref[...]
    conv = agg * dinv + b_ref[...]
    bn = (conv - m_ref[...]) * lax.rsqrt(v_ref[...] + 1e-5) * g_ref[...] + be_ref[...]
    return _gelu(bn) + res_ref[...]


def _tc2_body(acc_ref, hw_ref, res_ref, degp_ref, b_ref, g_ref, be_ref,
              m_ref, v_ref, w_ref, h_ref, hwn_ref):
    dinv = _dinv_of(degp_ref)
    h = _post_conv(acc_ref, hw_ref, res_ref, dinv, b_ref, g_ref, be_ref, m_ref, v_ref)
    h_ref[...] = h
    hwn_ref[...] = jnp.dot(h, w_ref[...], preferred_element_type=jnp.float32) * dinv


def _tc3_body(acc_ref, hw_ref, res_ref, degp_ref, b_ref, g_ref, be_ref,
              m_ref, v_ref, wlin_ref, blin_ref, out_ref):
    dinv = _dinv_of(degp_ref)
    h = _post_conv(acc_ref, hw_ref, res_ref, dinv, b_ref, g_ref, be_ref, m_ref, v_ref)
    out_ref[...] = jnp.dot(h, wlin_ref[...],
                           preferred_element_type=jnp.float32) + blin_ref[...]


_row_spec = pl.BlockSpec((BR, HD), lambda i: (i, 0))
_w_spec = pl.BlockSpec((HD, HD), lambda i: (0, 0))
_vec_spec = pl.BlockSpec((1, HD), lambda i: (0, 0))
_deg_spec = pl.BlockSpec((2, BR, DW), lambda i: (0, i, 0))
_acc_spec = pl.BlockSpec((2, BR, HD), lambda i: (0, i, 0))
_G = N // BR


def _tc0(x, wred, bred):
    return pl.pallas_call(
        _tc0_body,
        grid=(_G,),
        in_specs=[_row_spec, _w_spec, _vec_spec],
        out_specs=[_row_spec],
        out_shape=[jax.ShapeDtypeStruct((N, HD), jnp.float32)],
    )(x, wred, bred)[0]


def _tc1(h0, w1, degp):
    return pl.pallas_call(
        _tc1_body,
        grid=(_G,),
        in_specs=[_row_spec, _w_spec, _deg_spec],
        out_specs=[_row_spec],
        out_shape=[jax.ShapeDtypeStruct((N, HD), jnp.float32)],
    )(h0, w1, degp)[0]


def _tc2(acc, hw, res, degp, b, g, be, m, v, w):
    return pl.pallas_call(
        _tc2_body,
        grid=(_G,),
        in_specs=[_acc_spec, _row_spec, _row_spec, _deg_spec,
                  _vec_spec, _vec_spec, _vec_spec, _vec_spec, _vec_spec, _w_spec],
        out_specs=[_row_spec, _row_spec],
        out_shape=[jax.ShapeDtypeStruct((N, HD), jnp.float32)] * 2,
    )(acc, hw, res, degp, b, g, be, m, v, w)


def _tc3(acc, hw, res, degp, b, g, be, m, v, wlin, blin):
    return pl.pallas_call(
        _tc3_body,
        grid=(_G,),
        in_specs=[_acc_spec, _row_spec, _row_spec, _deg_spec,
                  _vec_spec, _vec_spec, _vec_spec, _vec_spec, _vec_spec,
                  pl.BlockSpec((HD, CLS), lambda i: (0, 0)),
                  pl.BlockSpec((1, CLS), lambda i: (0, 0))],
        out_specs=[pl.BlockSpec((BR, CLS), lambda i: (i, 0))],
        out_shape=[jax.ShapeDtypeStruct((N, CLS), jnp.float32)],
    )(acc, hw, res, degp, b, g, be, m, v, wlin, blin)[0]


def kernel(x, edge_index, W_red, b_red, W1, b1, g1, beta1, m1, v1,
           W2, b2, g2, beta2, m2, v2, W_lin, b_lin):
    src = edge_index[0]
    dst = edge_index[1]
    sidx = jnp.concatenate([src, jnp.zeros((EP - E,), jnp.int32)])
    didx = jnp.concatenate([dst, jnp.full((EP - E,), GR, jnp.int32)])
    sidx = sidx.reshape(NTILE, NJ, CH)
    didx = didx.reshape(NTILE, NJ, CH)

    h0 = _tc0(x, W_red, b_red.reshape(1, HD))
    degp = _deg_sc(didx)
    hw1 = _tc1(h0, W1, degp)
    acc1 = _gcn_agg(hw1, sidx, didx)
    h1, hw2 = _tc2(acc1, hw1, h0, degp, b1.reshape(1, HD), g1.reshape(1, HD),
                   beta1.reshape(1, HD), m1.reshape(1, HD), v1.reshape(1, HD), W2)
    acc2 = _gcn_agg(hw2, sidx, didx)
    return _tc3(acc2, hw2, h1, degp, b2.reshape(1, HD), g2.reshape(1, HD),
                beta2.reshape(1, HD), m2.reshape(1, HD), v2.reshape(1, HD),
                W_lin, b_lin.reshape(1, CLS))


# speedup vs baseline: 1.0100x; 1.0021x over previous
import functools

import jax
import jax.numpy as jnp
from jax import lax
from jax.experimental import pallas as pl
from jax.experimental.pallas import tpu as pltpu
from jax.experimental.pallas import tpu_sc as plsc

N = 10000
HD = 128
CLS = 40
E = 320000

NC = 2
NS = 16
NTILE = NC * NS
CH = 128
NJ = 80
EPT = NJ * CH
EP = NTILE * EPT
GR = N
NACC = 10240
SLAB = NACC // NS
TSL = 624
DW = 16
NBUF = 2
RB = 4

BR = 2000
_mesh = plsc.VectorSubcoreMesh(core_axis_name="core", subcore_axis_name="subcore")


@functools.partial(
    pl.kernel,
    out_type=jax.ShapeDtypeStruct((NC, NACC, DW), jnp.float32),
    mesh=_mesh,
    scratch_types=[
        pltpu.VMEM((NJ, CH), jnp.int32),
        pltpu.VMEM((CH, DW), jnp.float32),
        pltpu.VMEM((CH, DW), jnp.float32),
        pltpu.VMEM_SHARED((NACC, DW), jnp.float32),
        pltpu.SemaphoreType.DMA,
    ],
)
def _deg_sc(didx_hbm, out_hbm, didx_v, ones_v, zero_v, acc_sh, isem):
    c = lax.axis_index("core")
    s = lax.axis_index("subcore")
    wid = c * NS + s

    pltpu.async_copy(didx_hbm.at[wid], didx_v, isem)

    @pl.loop(0, CH)
    def _(r):
        ones_v[r, :] = jnp.ones((DW,), jnp.float32)
        zero_v[r, :] = jnp.zeros((DW,), jnp.float32)

    base = s * SLAB

    @pl.loop(0, SLAB // CH)
    def _(k):
        pltpu.sync_copy(zero_v, acc_sh.at[pl.ds(base + k * CH, CH)])

    pltpu.make_async_copy(didx_hbm.at[wid], didx_v, isem).wait()
    plsc.subcore_barrier()

    @pl.loop(0, NJ)
    def _(j):
        pltpu.sync_copy(ones_v, acc_sh.at[didx_v.at[j]], add=True)

    plsc.subcore_barrier()
    pltpu.sync_copy(acc_sh.at[pl.ds(base, SLAB)], out_hbm.at[c, pl.ds(base, SLAB)])


@functools.partial(
    pl.kernel,
    out_type=(jax.ShapeDtypeStruct((EP, HD), jnp.float32),
              jax.ShapeDtypeStruct((NC, NACC, HD), jnp.float32)),
    mesh=_mesh,
    scratch_types=[
        pltpu.VMEM((NJ, CH), jnp.int32),
        pltpu.VMEM((RB, CH), jnp.int32),
        pltpu.VMEM((RB, CH), jnp.int32),
        [pltpu.VMEM((CH, HD), jnp.float32)] * NBUF,
        pltpu.VMEM_SHARED((NACC, HD), jnp.float32),
        [pltpu.SemaphoreType.DMA] * NBUF,
        [pltpu.SemaphoreType.DMA] * NBUF,
        pltpu.SemaphoreType.DMA,
    ],
)
def _gcn_conv_sc(table_hbm, pidx_hbm, msg_hbm, out_hbm,
                 pidx_v, sring, dring, bufs, scr_sh, gsems, wsems, isem):
    c = lax.axis_index("core")
    s = lax.axis_index("subcore")
    wid = c * NS + s
    ebase = wid * EPT

    pltpu.async_copy(pidx_hbm.at[wid], pidx_v, isem)

    tb = s * TSL
    pltpu.sync_copy(table_hbm.at[pl.ds(tb, TSL)], scr_sh.at[pl.ds(tb, TSL)])

    @pl.when(s == 0)
    def _():
        pltpu.sync_copy(table_hbm.at[pl.ds(NS * TSL, N - NS * TSL)],
                        scr_sh.at[pl.ds(NS * TSL, N - NS * TSL)])

    pltpu.make_async_copy(pidx_hbm.at[wid], pidx_v, isem).wait()
    plsc.subcore_barrier()

    def _unpack_s(j):
        r = lax.rem(j, RB)

        @pl.loop(0, CH, step=16)
        def _(col):
            v = pidx_v[j, pl.ds(col, 16)]
            sring[r, pl.ds(col, 16)] = v & 0xFFFF

    def _unpack_d(j):
        r = lax.rem(j, RB)

        @pl.loop(0, CH, step=16)
        def _(col):
            v = pidx_v[j, pl.ds(col, 16)]
            dring[r, pl.ds(col, 16)] = lax.shift_right_logical(v, 16)

    def _gather(j, b):
        pltpu.async_copy(scr_sh.at[sring.at[lax.rem(j, RB)]], bufs[b], gsems[b])

    def _wait_gather(j, b):
        pltpu.make_async_copy(scr_sh.at[sring.at[lax.rem(j, RB)]],
                              bufs[b], gsems[b]).wait()

    def _write(j, b):
        pltpu.async_copy(bufs[b], msg_hbm.at[pl.ds(ebase + j * CH, CH)], wsems[b])

    def _wait_write(j, b):
        pltpu.make_async_copy(bufs[b], msg_hbm.at[pl.ds(ebase + j * CH, CH)],
                              wsems[b]).wait()

    _unpack_s(0)
    _gather(0, 0)

    @pl.loop(0, NJ, step=NBUF)
    def _(j):
        for b in range(NBUF):
            jj = j + b
            nb = (b + 1) % NBUF

            @pl.when(jj >= 1)
            def _():
                _wait_write(jj - 1, nb)

            @pl.when(jj + 1 < NJ)
            def _():
                _unpack_s(jj + 1)
                _gather(jj + 1, nb)

            _wait_gather(jj, b)
            _write(jj, b)

    _wait_write(NJ - 1, (NJ - 1) % NBUF)
    plsc.subcore_barrier()

    @pl.loop(0, CH)
    def _(r):
        @pl.loop(0, HD, step=16)
        def _(col):
            bufs[0][r, pl.ds(col, 16)] = jnp.zeros((16,), jnp.float32)

    base = s * SLAB

    @pl.loop(0, SLAB // CH)
    def _(k):
        pltpu.sync_copy(bufs[0], scr_sh.at[pl.ds(base + k * CH, CH)])

    plsc.subcore_barrier()

    def _read(j, b):
        pltpu.async_copy(msg_hbm.at[pl.ds(ebase + j * CH, CH)], bufs[b], gsems[b])

    def _wait_read(j, b):
        pltpu.make_async_copy(msg_hbm.at[pl.ds(ebase + j * CH, CH)],
                              bufs[b], gsems[b]).wait()

    def _scatter(j, b):
        pltpu.sync_copy(bufs[b], scr_sh.at[dring.at[lax.rem(j, RB)]], add=True)

    _unpack_d(0)
    _read(0, 0)

    @pl.loop(0, NJ, step=NBUF)
    def _(j):
        for b in range(NBUF):
            jj = j + b
            nb = (b + 1) % NBUF

            @pl.when(jj + 1 < NJ)
            def _():
                _unpack_d(jj + 1)
                _read(jj + 1, nb)

            _wait_read(jj, b)
            _scatter(jj, b)

    plsc.subcore_barrier()
    pltpu.sync_copy(scr_sh.at[pl.ds(base, SLAB)], out_hbm.at[c, pl.ds(base, SLAB)])


def _gcn_agg(hw, pidx):
    return _gcn_conv_sc(hw, pidx)[1]


def _gelu(x):
    return 0.5 * x * (1.0 + lax.erf(x * 0.7071067811865476))


def _dinv_of(degp_ref):
    deg = degp_ref[0] + degp_ref[1] + 1.0
    return lax.rsqrt(deg[:, 0:1])


def _tc0_body(x_ref, wred_ref, bred_ref, h0_ref):
    h0_ref[...] = _gelu(jnp.dot(x_ref[...], wred_ref[...],
                                preferred_element_type=jnp.float32) + bred_ref[...])


def _tc1_body(h0_ref, w1_ref, degp_ref, hw1_ref):
    dinv = _dinv_of(degp_ref)
    hw1_ref[...] = jnp.dot(h0_ref[...], w1_ref[...],
                           preferred_element_type=jnp.float32) * dinv


def _post_conv(acc_ref, hw_ref, res_ref, dinv, b_ref, g_ref, be_ref, m_ref, v_ref):
    agg = acc_ref[0] + acc_ref[1] + hw_ref[...]
    conv = agg * dinv + b_ref[...]
    bn = (conv - m_ref[...]) * lax.rsqrt(v_ref[...] + 1e-5) * g_ref[...] + be_ref[...]
    return _gelu(bn) + res_ref[...]


def _tc2_body(acc_ref, hw_ref, res_ref, degp_ref, b_ref, g_ref, be_ref,
              m_ref, v_ref, w_ref, h_ref, hwn_ref):
    dinv = _dinv_of(degp_ref)
    h = _post_conv(acc_ref, hw_ref, res_ref, dinv, b_ref, g_ref, be_ref, m_ref, v_ref)
    h_ref[...] = h
    hwn_ref[...] = jnp.dot(h, w_ref[...], preferred_element_type=jnp.float32) * dinv


def _tc3_body(acc_ref, hw_ref, res_ref, degp_ref, b_ref, g_ref, be_ref,
              m_ref, v_ref, wlin_ref, blin_ref, out_ref):
    dinv = _dinv_of(degp_ref)
    h = _post_conv(acc_ref, hw_ref, res_ref, dinv, b_ref, g_ref, be_ref, m_ref, v_ref)
    out_ref[...] = jnp.dot(h, wlin_ref[...],
                           preferred_element_type=jnp.float32) + blin_ref[...]


_row_spec = pl.BlockSpec((BR, HD), lambda i: (i, 0))
_w_spec = pl.BlockSpec((HD, HD), lambda i: (0, 0))
_vec_spec = pl.BlockSpec((1, HD), lambda i: (0, 0))
_deg_spec = pl.BlockSpec((2, BR, DW), lambda i: (0, i, 0))
_acc_spec = pl.BlockSpec((2, BR, HD), lambda i: (0, i, 0))
_G = N // BR


def _tc0(x, wred, bred):
    return pl.pallas_call(
        _tc0_body,
        grid=(_G,),
        in_specs=[_row_spec, _w_spec, _vec_spec],
        out_specs=[_row_spec],
        out_shape=[jax.ShapeDtypeStruct((N, HD), jnp.float32)],
    )(x, wred, bred)[0]


def _tc1(h0, w1, degp):
    return pl.pallas_call(
        _tc1_body,
        grid=(_G,),
        in_specs=[_row_spec, _w_spec, _deg_spec],
        out_specs=[_row_spec],
        out_shape=[jax.ShapeDtypeStruct((N, HD), jnp.float32)],
    )(h0, w1, degp)[0]


def _tc2(acc, hw, res, degp, b, g, be, m, v, w):
    return pl.pallas_call(
        _tc2_body,
        grid=(_G,),
        in_specs=[_acc_spec, _row_spec, _row_spec, _deg_spec,
                  _vec_spec, _vec_spec, _vec_spec, _vec_spec, _vec_spec, _w_spec],
        out_specs=[_row_spec, _row_spec],
        out_shape=[jax.ShapeDtypeStruct((N, HD), jnp.float32)] * 2,
    )(acc, hw, res, degp, b, g, be, m, v, w)


def _tc3(acc, hw, res, degp, b, g, be, m, v, wlin, blin):
    return pl.pallas_call(
        _tc3_body,
        grid=(_G,),
        in_specs=[_acc_spec, _row_spec, _row_spec, _deg_spec,
                  _vec_spec, _vec_spec, _vec_spec, _vec_spec, _vec_spec,
                  pl.BlockSpec((HD, CLS), lambda i: (0, 0)),
                  pl.BlockSpec((1, CLS), lambda i: (0, 0))],
        out_specs=[pl.BlockSpec((BR, CLS), lambda i: (i, 0))],
        out_shape=[jax.ShapeDtypeStruct((N, CLS), jnp.float32)],
    )(acc, hw, res, degp, b, g, be, m, v, wlin, blin)[0]


def kernel(x, edge_index, W_red, b_red, W1, b1, g1, beta1, m1, v1,
           W2, b2, g2, beta2, m2, v2, W_lin, b_lin):
    src = edge_index[0]
    dst = edge_index[1]
    sidx = jnp.concatenate([src, jnp.zeros((EP - E,), jnp.int32)])
    didx = jnp.concatenate([dst, jnp.full((EP - E,), GR, jnp.int32)])
    pidx = (sidx | (didx << 16)).reshape(NTILE, NJ, CH)
    didx = didx.reshape(NTILE, NJ, CH)

    h0 = _tc0(x, W_red, b_red.reshape(1, HD))
    degp = _deg_sc(didx)
    hw1 = _tc1(h0, W1, degp)
    acc1 = _gcn_agg(hw1, pidx)
    h1, hw2 = _tc2(acc1, hw1, h0, degp, b1.reshape(1, HD), g1.reshape(1, HD),
                   beta1.reshape(1, HD), m1.reshape(1, HD), v1.reshape(1, HD), W2)
    acc2 = _gcn_agg(hw2, pidx)
    return _tc3(acc2, hw2, h1, degp, b2.reshape(1, HD), g2.reshape(1, HD),
                beta2.reshape(1, HD), m2.reshape(1, HD), v2.reshape(1, HD),
                W_lin, b_lin.reshape(1, CLS))
